# baseline (speedup 1.0000x reference)
import functools

import jax
import jax.numpy as jnp
import numpy as np
from jax import lax
from jax.experimental import pallas as pl
from jax.experimental.pallas import tpu as pltpu
from jax.experimental.pallas import tpu_sc as plsc

NG = 10
NPG = 1000
NN = NG * NPG
NE = 320000
DN = 128
DE = 16
DM = 128
NL = 3
SPG = 500
NP = NG * SPG * 2

NW = 32
NNPAD = 10240
EC = 80
NCHUNK = 126
EPW = EC * NCHUNK
NEPAD = NW * EPW
QBLK = 8064

_CM = np.empty((DM,), np.int32)
for _k in range(DM // 32):
    for _i in range(16):
        _CM[32 * _k + 2 * _i] = 32 * _k + _i
        _CM[32 * _k + 2 * _i + 1] = 32 * _k + 16 + _i

RPS = NNPAD // 16
NPPAD = NW * 320
GC = 80


def _sc_mesh():
    return plsc.VectorSubcoreMesh(core_axis_name="c", subcore_axis_name="s")


def _weights_body(we, be, wm, bm, ke_o, cb_o):
    for l in range(NL):
        ke_o[l] = jnp.dot(we[...], wm[l], preferred_element_type=jnp.float32)
        cb_o[l] = jnp.dot(be[...], wm[l], preferred_element_type=jnp.float32) + bm[l]


def _weights_call(We, be2, Wm, bm3):
    return pl.pallas_call(
        _weights_body,
        out_shape=(
            jax.ShapeDtypeStruct((NL, DE, DM), jnp.float32),
            jax.ShapeDtypeStruct((NL, 1, DM), jnp.float32),
        ),
    )(We, be2, Wm, bm3)


def _enc_body(xb, wn, bn, wm0, cb0, nf_o, p_o):
    nf = jnp.dot(xb[...], wn[...], preferred_element_type=jnp.float32) + bn[...]
    nf_o[...] = nf
    p_o[...] = jnp.dot(nf, wm0[...], preferred_element_type=jnp.float32) + cb0[...]


def _enc_call(x, Wn, bn2, Wm0, cb0):
    blk = 1000
    grid = NN // blk
    return pl.pallas_call(
        _enc_body,
        grid=(grid,),
        in_specs=[
            pl.BlockSpec((blk, DN), lambda i: (i, 0)),
            pl.BlockSpec((DN, DM), lambda i: (0, 0)),
            pl.BlockSpec((1, DM), lambda i: (0, 0)),
            pl.BlockSpec((DM, DM), lambda i: (0, 0)),
            pl.BlockSpec((1, DM), lambda i: (0, 0)),
        ],
        out_specs=(
            pl.BlockSpec((blk, DM), lambda i: (i, 0)),
            pl.BlockSpec((blk, DM), lambda i: (i, 0)),
        ),
        out_shape=(
            jax.ShapeDtypeStruct((NN, DM), jnp.float32),
            jax.ShapeDtypeStruct((NN, DM), jnp.float32),
        ),
    )(x, Wn, bn2, Wm0, cb0)


def _q_body(ab, ke, q_o):
    q = jnp.dot(ab[...], ke[...], preferred_element_type=jnp.float32)
    q_o[...] = q.astype(jnp.bfloat16)


def _q_call(edge_attr_pad, Ke_l):
    grid = NEPAD // QBLK
    return pl.pallas_call(
        _q_body,
        grid=(grid,),
        in_specs=[
            pl.BlockSpec((QBLK, DE), lambda i: (i, 0)),
            pl.BlockSpec((DE, DM), lambda i: (0, 0)),
        ],
        out_specs=pl.BlockSpec((QBLK, DM), lambda i: (i, 0)),
        out_shape=jax.ShapeDtypeStruct((NEPAD, DM), jnp.bfloat16),
    )(edge_attr_pad, Ke_l)


def _upd_body(nfb, ab, wu, bu, wmn, cbn, nf_o, p_o):
    aggr = ab[0] + ab[1]
    t = jnp.dot(nfb[...] + aggr, wu[...], preferred_element_type=jnp.float32) + bu[...]
    nf2 = nfb[...] + jnp.maximum(t, 0.0)
    nf_o[...] = nf2
    p_o[...] = jnp.dot(nf2, wmn[...], preferred_element_type=jnp.float32) + cbn[...]


def _upd_call(nf, aggr2, Wu_l, bu2, Wm_n, cb_n):
    blk = 1000
    grid = NN // blk
    return pl.pallas_call(
        _upd_body,
        grid=(grid,),
        in_specs=[
            pl.BlockSpec((blk, DM), lambda i: (i, 0)),
            pl.BlockSpec((2, blk, DM), lambda i: (0, i, 0)),
            pl.BlockSpec((DM, DM), lambda i: (0, 0)),
            pl.BlockSpec((1, DM), lambda i: (0, 0)),
            pl.BlockSpec((DM, DM), lambda i: (0, 0)),
            pl.BlockSpec((1, DM), lambda i: (0, 0)),
        ],
        out_specs=(
            pl.BlockSpec((blk, DM), lambda i: (i, 0)),
            pl.BlockSpec((blk, DM), lambda i: (i, 0)),
        ),
        out_shape=(
            jax.ShapeDtypeStruct((NN, DM), jnp.float32),
            jax.ShapeDtypeStruct((NN, DM), jnp.float32),
        ),
    )(nf, aggr2, Wu_l, bu2, Wm_n, cb_n)


def _mlp_body(h, yb, w1, b1, w2, b2, w3, b3, o):
    a = jnp.maximum(jnp.dot(h[...], w1[...], preferred_element_type=jnp.float32) + b1[...], 0.0)
    b = jnp.maximum(jnp.dot(a, w2[...], preferred_element_type=jnp.float32) + b2[...], 0.0)
    yh = jnp.dot(b, w3[...], preferred_element_type=jnp.float32) + b3[...]
    d = yh - yb[...]
    o[...] = (jnp.sum(d * d) / (NG * SPG)).reshape(1, 1)


def _mlp_call(h, y, W1, b12, W2, b22, W3, b32):
    return pl.pallas_call(
        _mlp_body,
        out_shape=jax.ShapeDtypeStruct((1, 1), jnp.float32),
    )(h, y, W1, b12, W2, b22, W3, b32)


def _edge_body(p_hbm, q_hbm, src_hbm, dst_hbm, out_hbm,
               src0, src1, src2, dst0, dst1, dst2, dst3, dst4, dst5,
               q0, q1, q2, prw0, prw1, prw2, aggr_sh,
               l0, l1, l2, g0, g1, g2, s0, s1, s2):
    c = lax.axis_index("c")
    s = lax.axis_index("s")
    srcb = (src0, src1, src2)
    dstb = (dst0, dst1, dst2, dst3, dst4, dst5)
    qb = (q0, q1, q2)
    prwb = (prw0, prw1, prw2)
    lsem = (l0, l1, l2)
    gsem = (g0, g1, g2)
    ssem = (s0, s1, s2)

    zero16 = jnp.zeros((16,), jnp.float32)

    def zrow(r, carry):
        for j in range(DM // 16):
            prw0[r, pl.ds(j * 16, 16)] = zero16
        return carry

    lax.fori_loop(0, EC, zrow, 0)
    for k in range(RPS // EC):
        pltpu.sync_copy(prw0, aggr_sh.at[pl.ds(s * RPS + k * EC, EC)])
    plsc.subcore_barrier()

    base = (c * 16 + s) * EPW

    def start_load(p3, p6, ci):
        off = base + ci * EC
        pltpu.async_copy(src_hbm.at[pl.ds(off, EC)], srcb[p3], lsem[p3])
        pltpu.async_copy(dst_hbm.at[pl.ds(off, EC)], dstb[p6], lsem[p3])
        pltpu.async_copy(
            q_hbm.at[pl.ds(off * (DM // 2), EC * (DM // 2))], qb[p3], lsem[p3])

    def wait_load(p3, p6, ci):
        off = base + ci * EC
        pltpu.make_async_copy(src_hbm.at[pl.ds(off, EC)], srcb[p3], lsem[p3]).wait()
        pltpu.make_async_copy(dst_hbm.at[pl.ds(off, EC)], dstb[p6], lsem[p3]).wait()
        pltpu.make_async_copy(
            q_hbm.at[pl.ds(off * (DM // 2), EC * (DM // 2))], qb[p3],
            lsem[p3]).wait()

    def start_gather(p3):
        pltpu.async_copy(p_hbm.at[srcb[p3]], prwb[p3], gsem[p3])

    def wait_gather(p3):
        pltpu.make_async_copy(p_hbm.at[srcb[p3]], prwb[p3], gsem[p3]).wait()

    def start_scatter(p3, p6):
        pltpu.async_copy(prwb[p3], aggr_sh.at[dstb[p6]], ssem[p3], add=True)

    def wait_scatter(p3, p6):
        pltpu.make_async_copy(prwb[p3], aggr_sh.at[dstb[p6]], ssem[p3]).wait()

    def compute(p3):
        q_v, prw_v = qb[p3], prwb[p3]
        mask_hi = jnp.full((16,), -65536, jnp.int32)

        def row(r, rc):
            qoff = r * (DM // 2)
            for j in range(DM // 32):
                qi = q_v[pl.ds(qoff + j * 16, 16)]
                ql = jax.lax.bitcast_convert_type(qi << 16, jnp.float32)
                qh = jax.lax.bitcast_convert_type(qi & mask_hi, jnp.float32)
                sl_lo = pl.ds(j * 32, 16)
                sl_hi = pl.ds(j * 32 + 16, 16)
                prw_v[r, sl_lo] = jnp.maximum(prw_v[r, sl_lo] + ql, 0.0)
                prw_v[r, sl_hi] = jnp.maximum(prw_v[r, sl_hi] + qh, 0.0)
            return rc

        lax.fori_loop(0, EC, row, 0)

    start_load(0, 0, 0)
    start_load(1, 1, 1)
    start_load(2, 2, 2)
    wait_load(0, 0, 0)
    start_gather(0)

    def six_chunks(t, carry):
        c0 = 6 * t
        for k in range(6):
            ci = c0 + k
            p3 = k % 3
            if k < 5:
                wait_load((k + 1) % 3, (k + 1) % 6, ci + 1)
            else:
                @pl.when(ci + 1 < NCHUNK)
                def _():
                    wait_load(0, 0, ci + 1)

            @pl.when(ci >= 2)
            def _():
                wait_scatter((k + 1) % 3, (k + 4) % 6)

            if k < 5:
                start_gather((k + 1) % 3)
            else:
                @pl.when(ci + 1 < NCHUNK)
                def _():
                    start_gather(0)
            wait_gather(p3)
            compute(p3)
            start_scatter(p3, k)
            if k < 3:
                start_load(p3, (k + 3) % 6, ci + 3)
            else:
                @pl.when(ci + 3 < NCHUNK)
                def _():
                    start_load(p3, (k + 3) % 6, ci + 3)
        return carry

    lax.fori_loop(0, NCHUNK // 6, six_chunks, 0)
    wait_scatter((NCHUNK - 2) % 3, 4)
    wait_scatter((NCHUNK - 1) % 3, 5)
    plsc.subcore_barrier()
    for k in range(RPS // EC):
        r0 = s * RPS + k * EC
        pltpu.sync_copy(aggr_sh.at[pl.ds(r0, EC)], out_hbm.at[c, pl.ds(r0, EC)])


@functools.partial(
    pl.kernel,
    out_type=jax.ShapeDtypeStruct((2, NNPAD, DM), jnp.float32),
    mesh=_sc_mesh(),
    scratch_types=[
        pltpu.VMEM((EC,), jnp.int32),
        pltpu.VMEM((EC,), jnp.int32),
        pltpu.VMEM((EC,), jnp.int32),
        pltpu.VMEM((EC,), jnp.int32),
        pltpu.VMEM((EC,), jnp.int32),
        pltpu.VMEM((EC,), jnp.int32),
        pltpu.VMEM((EC,), jnp.int32),
        pltpu.VMEM((EC,), jnp.int32),
        pltpu.VMEM((EC,), jnp.int32),
        pltpu.VMEM((EC * (DM // 2),), jnp.int32),
        pltpu.VMEM((EC * (DM // 2),), jnp.int32),
        pltpu.VMEM((EC * (DM // 2),), jnp.int32),
        pltpu.VMEM((EC, DM), jnp.float32),
        pltpu.VMEM((EC, DM), jnp.float32),
        pltpu.VMEM((EC, DM), jnp.float32),
        pltpu.VMEM_SHARED((NNPAD, DM), jnp.float32),
        pltpu.SemaphoreType.DMA,
        pltpu.SemaphoreType.DMA,
        pltpu.SemaphoreType.DMA,
        pltpu.SemaphoreType.DMA,
        pltpu.SemaphoreType.DMA,
        pltpu.SemaphoreType.DMA,
        pltpu.SemaphoreType.DMA,
        pltpu.SemaphoreType.DMA,
        pltpu.SemaphoreType.DMA,
    ],
)
def _edge_call(p_hbm, q_hbm, src_hbm, dst_hbm, out_hbm,
               src0, src1, src2, dst0, dst1, dst2, dst3, dst4, dst5,
               q0, q1, q2, prw0, prw1, prw2, aggr_sh,
               l0, l1, l2, g0, g1, g2, s0, s1, s2):
    _edge_body(p_hbm, q_hbm, src_hbm, dst_hbm, out_hbm,
               src0, src1, src2, dst0, dst1, dst2, dst3, dst4, dst5,
               q0, q1, q2, prw0, prw1, prw2, aggr_sh,
               l0, l1, l2, g0, g1, g2, s0, s1, s2)


@functools.partial(
    pl.kernel,
    out_type=jax.ShapeDtypeStruct((NPPAD, DM), jnp.float32),
    mesh=_sc_mesh(),
    scratch_types=[
        pltpu.VMEM((GC,), jnp.int32),
        pltpu.VMEM((GC, DM), jnp.float32),
        pltpu.SemaphoreType.DMA,
    ],
)
def _pair_gather(tbl_hbm, idx_hbm, out_hbm, idx_v, row_v, sem):
    c = lax.axis_index("c")
    s = lax.axis_index("s")
    base = (c * 16 + s) * (NPPAD // NW)

    def chunk(i, carry):
        off = base + i * GC
        pltpu.sync_copy(idx_hbm.at[pl.ds(off, GC)], idx_v)
        pltpu.async_copy(tbl_hbm.at[idx_v], row_v, sem).wait()
        pltpu.sync_copy(row_v, out_hbm.at[pl.ds(off, GC)])
        return carry

    lax.fori_loop(0, (NPPAD // NW) // GC, chunk, 0)


def kernel(x, edge_index, edge_attr, schedule, y,
           Wn, bn, We, be, Wm, bm, Wu, bu, W1, b1, W2, b2, W3, b3):
    npad = NEPAD - NE
    src = jnp.concatenate([edge_index[0], jnp.zeros((npad,), jnp.int32)])
    dst = jnp.concatenate(
        [edge_index[1], jnp.full((npad,), NNPAD - 1, jnp.int32)])
    ea_pad = jnp.concatenate(
        [edge_attr, jnp.zeros((npad, DE), jnp.float32)])
    bn2 = bn.reshape(1, DM)
    be2 = be.reshape(1, DM)
    bm3 = bm.reshape(NL, 1, DM)

    def _as_i32_flat(a):
        return jax.lax.bitcast_convert_type(
            a.reshape(a.shape[0], DM // 2, 2), jnp.int32).reshape(-1)

    Ke, cb = _weights_call(We, be2, Wm, bm3)
    Ke = Ke[:, :, _CM]
    nf, P = _enc_call(x, Wn, bn2, Wm[0], cb[0])
    Qs = [_as_i32_flat(_q_call(ea_pad, Ke[l])) for l in range(NL)]
    for l in range(NL):
        aggr2 = _edge_call(P, Qs[l], src, dst)
        nf, P = _upd_call(nf, aggr2, Wu[l], bu[l].reshape(1, DM),
                          Wm[(l + 1) % NL], cb[(l + 1) % NL])

    offs = (jnp.arange(NG, dtype=jnp.int32) * NPG)[:, None]
    flat = (schedule.reshape(NG, SPG * 2) + offs).reshape(-1)
    flat = jnp.concatenate(
        [flat, jnp.zeros((NPPAD - NP,), jnp.int32)])
    pairs = _pair_gather(nf, flat)
    h = pairs[:NP].reshape(NG * SPG, 2 * DM)

    loss = _mlp_call(h, y, W1, b1.reshape(1, DM), W2, b2.reshape(1, DM // 2),
                     W3, b3.reshape(1, 1))
    return loss.reshape(())


# speedup vs baseline: 3.0718x; 3.0718x over previous
import functools

import jax
import jax.numpy as jnp
import numpy as np
from jax import lax
from jax.experimental import pallas as pl
from jax.experimental.pallas import tpu as pltpu
from jax.experimental.pallas import tpu_sc as plsc

NG = 10
NPG = 1000
NN = NG * NPG
NE = 320000
DN = 128
DE = 16
DM = 128
NL = 3
SPG = 500
NP = NG * SPG * 2

NW = 32
NNPAD = 10240
EC = 64
NCHUNK = 162
EPW = EC * NCHUNK
NEPAD = NW * EPW
QBLK = 6912

RPS = NNPAD // 16
NPPAD = NW * 320
GC = 80


def _sc_mesh():
    return plsc.VectorSubcoreMesh(core_axis_name="c", subcore_axis_name="s")


def _weights_body(we, be, wm, bm, ke_o, cb_o):
    z = jnp.zeros((DE, DM), jnp.float32)
    for l in range(NL):
        ke = jnp.dot(we[...], wm[l], preferred_element_type=jnp.float32)
        ke_o[l] = jnp.concatenate(
            [jnp.concatenate([ke, z], axis=1),
             jnp.concatenate([z, ke], axis=1)], axis=0)
        cb_o[l] = jnp.dot(be[...], wm[l], preferred_element_type=jnp.float32) + bm[l]


def _weights_call(We, be2, Wm, bm3):
    return pl.pallas_call(
        _weights_body,
        out_shape=(
            jax.ShapeDtypeStruct((NL, 2 * DE, 2 * DM), jnp.float32),
            jax.ShapeDtypeStruct((NL, 1, DM), jnp.float32),
        ),
    )(We, be2, Wm, bm3)


def _enc_body(xb, wn, bn, wm0, cb0, nf_o, p_o):
    nf = jnp.dot(xb[...], wn[...], preferred_element_type=jnp.float32) + bn[...]
    nf_o[...] = nf
    p_o[...] = jnp.dot(nf, wm0[...], preferred_element_type=jnp.float32) + cb0[...]


def _enc_call(x, Wn, bn2, Wm0, cb0):
    blk = 1000
    grid = NN // blk
    return pl.pallas_call(
        _enc_body,
        grid=(grid,),
        in_specs=[
            pl.BlockSpec((blk, DN), lambda i: (i, 0)),
            pl.BlockSpec((DN, DM), lambda i: (0, 0)),
            pl.BlockSpec((1, DM), lambda i: (0, 0)),
            pl.BlockSpec((DM, DM), lambda i: (0, 0)),
            pl.BlockSpec((1, DM), lambda i: (0, 0)),
        ],
        out_specs=(
            pl.BlockSpec((blk, DM), lambda i: (i, 0)),
            pl.BlockSpec((blk, DM), lambda i: (i, 0)),
        ),
        out_shape=(
            jax.ShapeDtypeStruct((NN, DM), jnp.float32),
            jax.ShapeDtypeStruct((NN, DM), jnp.float32),
        ),
    )(x, Wn, bn2, Wm0, cb0)


def _q_body(ab, ke, q_o):
    t = jnp.dot(ab[...], ke[...], preferred_element_type=jnp.float32)
    tb = t.astype(jnp.bfloat16)
    au = jax.lax.bitcast_convert_type(tb[:, :DM], jnp.uint16).astype(jnp.int32)
    bu = jax.lax.bitcast_convert_type(tb[:, DM:], jnp.uint16).astype(jnp.int32)
    q_o[...] = au | (bu << 16)


def _q_call(edge_attr_pairs, Ke2_l):
    grid = (NEPAD // 2) // (QBLK // 2)
    return pl.pallas_call(
        _q_body,
        grid=(grid,),
        in_specs=[
            pl.BlockSpec((QBLK // 2, 2 * DE), lambda i: (i, 0)),
            pl.BlockSpec((2 * DE, 2 * DM), lambda i: (0, 0)),
        ],
        out_specs=pl.BlockSpec((QBLK // 2, DM), lambda i: (i, 0)),
        out_shape=jax.ShapeDtypeStruct((NEPAD // 2, DM), jnp.int32),
    )(edge_attr_pairs, Ke2_l)


def _upd_body(nfb, ab, wu, bu, wmn, cbn, nf_o, p_o):
    aggr = ab[0] + ab[1]
    t = jnp.dot(nfb[...] + aggr, wu[...], preferred_element_type=jnp.float32) + bu[...]
    nf2 = nfb[...] + jnp.maximum(t, 0.0)
    nf_o[...] = nf2
    p_o[...] = jnp.dot(nf2, wmn[...], preferred_element_type=jnp.float32) + cbn[...]


def _upd_call(nf, aggr2, Wu_l, bu2, Wm_n, cb_n):
    blk = 1000
    grid = NN // blk
    return pl.pallas_call(
        _upd_body,
        grid=(grid,),
        in_specs=[
            pl.BlockSpec((blk, DM), lambda i: (i, 0)),
            pl.BlockSpec((2, blk, DM), lambda i: (0, i, 0)),
            pl.BlockSpec((DM, DM), lambda i: (0, 0)),
            pl.BlockSpec((1, DM), lambda i: (0, 0)),
            pl.BlockSpec((DM, DM), lambda i: (0, 0)),
            pl.BlockSpec((1, DM), lambda i: (0, 0)),
        ],
        out_specs=(
            pl.BlockSpec((blk, DM), lambda i: (i, 0)),
            pl.BlockSpec((blk, DM), lambda i: (i, 0)),
        ),
        out_shape=(
            jax.ShapeDtypeStruct((NN, DM), jnp.float32),
            jax.ShapeDtypeStruct((NN, DM), jnp.float32),
        ),
    )(nf, aggr2, Wu_l, bu2, Wm_n, cb_n)


def _mlp_body(h, yb, w1, b1, w2, b2, w3, b3, o):
    a = jnp.maximum(jnp.dot(h[...], w1[...], preferred_element_type=jnp.float32) + b1[...], 0.0)
    b = jnp.maximum(jnp.dot(a, w2[...], preferred_element_type=jnp.float32) + b2[...], 0.0)
    yh = jnp.dot(b, w3[...], preferred_element_type=jnp.float32) + b3[...]
    d = yh - yb[...]
    o[...] = (jnp.sum(d * d) / (NG * SPG)).reshape(1, 1)


def _mlp_call(h, y, W1, b12, W2, b22, W3, b32):
    return pl.pallas_call(
        _mlp_body,
        out_shape=jax.ShapeDtypeStruct((1, 1), jnp.float32),
    )(h, y, W1, b12, W2, b22, W3, b32)


def _edge_body(p_hbm, q_hbm, src_hbm, dst_hbm, out_hbm,
               src0, src1, src2, dst0, dst1, dst2, dst3, dst4, dst5,
               q0, q1, q2, prw0, prw1, prw2, aggr_sh,
               l0, l1, l2, g0, g1, g2, s0, s1, s2):
    c = lax.axis_index("c")
    s = lax.axis_index("s")
    srcb = (src0, src1, src2)
    dstb = (dst0, dst1, dst2, dst3, dst4, dst5)
    qb = (q0, q1, q2)
    prwb = (prw0, prw1, prw2)
    lsem = (l0, l1, l2)
    gsem = (g0, g1, g2)
    ssem = (s0, s1, s2)

    zero16 = jnp.zeros((16,), jnp.float32)

    def zrow(r, carry):
        for j in range(DM // 16):
            prw0[r, pl.ds(j * 16, 16)] = zero16
        return carry

    lax.fori_loop(0, EC, zrow, 0)
    for k in range(RPS // EC):
        pltpu.sync_copy(prw0, aggr_sh.at[pl.ds(s * RPS + k * EC, EC)])
    plsc.subcore_barrier()

    base = (c * 16 + s) * EPW

    def start_load(p3, p6, ci):
        off = base + ci * EC
        pltpu.async_copy(src_hbm.at[pl.ds(off, EC)], srcb[p3], lsem[p3])
        pltpu.async_copy(dst_hbm.at[pl.ds(off, EC)], dstb[p6], lsem[p3])
        off2 = pl.multiple_of(off // 2, 8)
        pltpu.async_copy(q_hbm.at[pl.ds(off2, EC // 2)], qb[p3], lsem[p3])

    def wait_load(p3, p6, ci):
        off = base + ci * EC
        pltpu.make_async_copy(src_hbm.at[pl.ds(off, EC)], srcb[p3], lsem[p3]).wait()
        pltpu.make_async_copy(dst_hbm.at[pl.ds(off, EC)], dstb[p6], lsem[p3]).wait()
        off2 = pl.multiple_of(off // 2, 8)
        pltpu.make_async_copy(
            q_hbm.at[pl.ds(off2, EC // 2)], qb[p3], lsem[p3]).wait()

    def start_gather(p3):
        pltpu.async_copy(p_hbm.at[srcb[p3]], prwb[p3], gsem[p3])

    def wait_gather(p3):
        pltpu.make_async_copy(p_hbm.at[srcb[p3]], prwb[p3], gsem[p3]).wait()

    def start_scatter(p3, p6):
        pltpu.async_copy(prwb[p3], aggr_sh.at[dstb[p6]], ssem[p3], add=True)

    def wait_scatter(p3, p6):
        pltpu.make_async_copy(prwb[p3], aggr_sh.at[dstb[p6]], ssem[p3]).wait()

    def compute(p3):
        q_v, prw_v = qb[p3], prwb[p3]
        mask_hi = jnp.full((16,), -65536, jnp.int32)

        def rowpair(m, rc):
            r0 = 2 * m
            for j in range(DM // 16):
                sl = pl.ds(j * 16, 16)
                qi = q_v[m, sl]
                ql = jax.lax.bitcast_convert_type(qi << 16, jnp.float32)
                qh = jax.lax.bitcast_convert_type(qi & mask_hi, jnp.float32)
                prw_v[r0, sl] = jnp.maximum(prw_v[r0, sl] + ql, 0.0)
                prw_v[r0 + 1, sl] = jnp.maximum(prw_v[r0 + 1, sl] + qh, 0.0)
            return rc

        lax.fori_loop(0, EC // 2, rowpair, 0)

    start_load(0, 0, 0)
    start_load(1, 1, 1)
    start_load(2, 2, 2)
    wait_load(0, 0, 0)
    start_gather(0)

    def six_chunks(t, carry):
        c0 = 6 * t
        for k in range(6):
            ci = c0 + k
            p3 = k % 3
            if k < 5:
                wait_load((k + 1) % 3, (k + 1) % 6, ci + 1)
            else:
                @pl.when(ci + 1 < NCHUNK)
                def _():
                    wait_load(0, 0, ci + 1)

            @pl.when(ci >= 2)
            def _():
                wait_scatter((k + 1) % 3, (k + 4) % 6)

            if k < 5:
                start_gather((k + 1) % 3)
            else:
                @pl.when(ci + 1 < NCHUNK)
                def _():
                    start_gather(0)
            wait_gather(p3)
            compute(p3)
            start_scatter(p3, k)
            if k < 3:
                start_load(p3, (k + 3) % 6, ci + 3)
            else:
                @pl.when(ci + 3 < NCHUNK)
                def _():
                    start_load(p3, (k + 3) % 6, ci + 3)
        return carry

    lax.fori_loop(0, NCHUNK // 6, six_chunks, 0)
    wait_scatter((NCHUNK - 2) % 3, 4)
    wait_scatter((NCHUNK - 1) % 3, 5)
    plsc.subcore_barrier()
    for k in range(RPS // EC):
        r0 = s * RPS + k * EC
        pltpu.sync_copy(aggr_sh.at[pl.ds(r0, EC)], out_hbm.at[c, pl.ds(r0, EC)])


@functools.partial(
    pl.kernel,
    out_type=jax.ShapeDtypeStruct((2, NNPAD, DM), jnp.float32),
    mesh=_sc_mesh(),
    scratch_types=[
        pltpu.VMEM((EC,), jnp.int32),
        pltpu.VMEM((EC,), jnp.int32),
        pltpu.VMEM((EC,), jnp.int32),
        pltpu.VMEM((EC,), jnp.int32),
        pltpu.VMEM((EC,), jnp.int32),
        pltpu.VMEM((EC,), jnp.int32),
        pltpu.VMEM((EC,), jnp.int32),
        pltpu.VMEM((EC,), jnp.int32),
        pltpu.VMEM((EC,), jnp.int32),
        pltpu.VMEM((EC // 2, DM), jnp.int32),
        pltpu.VMEM((EC // 2, DM), jnp.int32),
        pltpu.VMEM((EC // 2, DM), jnp.int32),
        pltpu.VMEM((EC, DM), jnp.float32),
        pltpu.VMEM((EC, DM), jnp.float32),
        pltpu.VMEM((EC, DM), jnp.float32),
        pltpu.VMEM_SHARED((NNPAD, DM), jnp.float32),
        pltpu.SemaphoreType.DMA,
        pltpu.SemaphoreType.DMA,
        pltpu.SemaphoreType.DMA,
        pltpu.SemaphoreType.DMA,
        pltpu.SemaphoreType.DMA,
        pltpu.SemaphoreType.DMA,
        pltpu.SemaphoreType.DMA,
        pltpu.SemaphoreType.DMA,
        pltpu.SemaphoreType.DMA,
    ],
)
def _edge_call(p_hbm, q_hbm, src_hbm, dst_hbm, out_hbm,
               src0, src1, src2, dst0, dst1, dst2, dst3, dst4, dst5,
               q0, q1, q2, prw0, prw1, prw2, aggr_sh,
               l0, l1, l2, g0, g1, g2, s0, s1, s2):
    _edge_body(p_hbm, q_hbm, src_hbm, dst_hbm, out_hbm,
               src0, src1, src2, dst0, dst1, dst2, dst3, dst4, dst5,
               q0, q1, q2, prw0, prw1, prw2, aggr_sh,
               l0, l1, l2, g0, g1, g2, s0, s1, s2)


@functools.partial(
    pl.kernel,
    out_type=jax.ShapeDtypeStruct((NPPAD, DM), jnp.float32),
    mesh=_sc_mesh(),
    scratch_types=[
        pltpu.VMEM((GC,), jnp.int32),
        pltpu.VMEM((GC, DM), jnp.float32),
        pltpu.SemaphoreType.DMA,
    ],
)
def _pair_gather(tbl_hbm, idx_hbm, out_hbm, idx_v, row_v, sem):
    c = lax.axis_index("c")
    s = lax.axis_index("s")
    base = (c * 16 + s) * (NPPAD // NW)

    def chunk(i, carry):
        off = base + i * GC
        pltpu.sync_copy(idx_hbm.at[pl.ds(off, GC)], idx_v)
        pltpu.async_copy(tbl_hbm.at[idx_v], row_v, sem).wait()
        pltpu.sync_copy(row_v, out_hbm.at[pl.ds(off, GC)])
        return carry

    lax.fori_loop(0, (NPPAD // NW) // GC, chunk, 0)


def kernel(x, edge_index, edge_attr, schedule, y,
           Wn, bn, We, be, Wm, bm, Wu, bu, W1, b1, W2, b2, W3, b3):
    npad = NEPAD - NE
    pidx = jnp.arange(npad, dtype=jnp.int32)
    src = jnp.concatenate([edge_index[0], pidx % NN])
    dst = jnp.concatenate([edge_index[1], NN + (pidx % (NNPAD - NN))])
    ea_pad = jnp.concatenate(
        [edge_attr, jnp.zeros((npad, DE), jnp.float32)])
    bn2 = bn.reshape(1, DM)
    be2 = be.reshape(1, DM)
    bm3 = bm.reshape(NL, 1, DM)

    ea2 = ea_pad.reshape(NEPAD // 2, 2 * DE)

    Ke2, cb = _weights_call(We, be2, Wm, bm3)
    nf, P = _enc_call(x, Wn, bn2, Wm[0], cb[0])
    Qs = [_q_call(ea2, Ke2[l]) for l in range(NL)]
    for l in range(NL):
        aggr2 = _edge_call(P, Qs[l], src, dst)
        nf, P = _upd_call(nf, aggr2, Wu[l], bu[l].reshape(1, DM),
                          Wm[(l + 1) % NL], cb[(l + 1) % NL])

    offs = (jnp.arange(NG, dtype=jnp.int32) * NPG)[:, None]
    flat = (schedule.reshape(NG, SPG * 2) + offs).reshape(-1)
    flat = jnp.concatenate(
        [flat, jnp.zeros((NPPAD - NP,), jnp.int32)])
    pairs = _pair_gather(nf, flat)
    h = pairs[:NP].reshape(NG * SPG, 2 * DM)

    loss = _mlp_call(h, y, W1, b1.reshape(1, DM), W2, b2.reshape(1, DM // 2),
                     W3, b3.reshape(1, 1))
    return loss.reshape(())
